# Initial kernel scaffold; baseline (speedup 1.0000x reference)
import jax
import jax.numpy as jnp
from jax.experimental import pallas as pl


def kernel(pos, edge_index, batch, W1a, b1a, W1b, b1b, W2a, b2a, W2b, b2b):
    raise NotImplementedError("write your pallas kernel here")


# speedup vs baseline: 1.4392x; 1.4392x over previous
import jax
import jax.numpy as jnp
from jax import lax
from jax.experimental import pallas as pl
from jax.experimental.pallas import tpu as pltpu
from jax.experimental.pallas import tpu_sc as plsc

HID = 32
NW = 32
GC = 200
SCC = 2048
NBLK = 1024
EBLK = 7168
TW = 128


def _wid():
    return lax.axis_index("s") * 2 + lax.axis_index("c")


def _tables_body(pos, wg1, wd1, w2p, b1a, b2a, t1, p2, d2):
    p = pos[...]
    g1v = jax.lax.dot_general(p, wg1[...], (((1,), (0,)), ((), ())),
                              preferred_element_type=jnp.float32) + b1a[...]
    d1v = jax.lax.dot_general(p, wd1[...], (((1,), (0,)), ((), ())),
                              preferred_element_type=jnp.float32)
    d2v = jax.lax.dot_general(p, w2p[...], (((1,), (0,)), ((), ())),
                              preferred_element_type=jnp.float32)
    pad = jnp.zeros((p.shape[0], TW - 2 * HID), jnp.float32)
    t1[...] = jnp.concatenate([g1v, d1v, pad], axis=1)
    p2[...] = d2v + b2a[...]
    d2[...] = d2v


def _tc_tables(pos_pad, wg1, wd1, w2p, b1a, b2a):
    np_ = pos_pad.shape[0]
    grid = np_ // NBLK
    return pl.pallas_call(
        _tables_body,
        grid=(grid,),
        in_specs=[
            pl.BlockSpec((NBLK, 3), lambda i: (i, 0)),
            pl.BlockSpec((3, HID), lambda i: (0, 0)),
            pl.BlockSpec((3, HID), lambda i: (0, 0)),
            pl.BlockSpec((3, HID), lambda i: (0, 0)),
            pl.BlockSpec((1, HID), lambda i: (0, 0)),
            pl.BlockSpec((1, HID), lambda i: (0, 0)),
        ],
        out_specs=[
            pl.BlockSpec((NBLK, TW), lambda i: (i, 0)),
            pl.BlockSpec((NBLK, HID), lambda i: (i, 0)),
            pl.BlockSpec((NBLK, HID), lambda i: (i, 0)),
        ],
        out_shape=[
            jax.ShapeDtypeStruct((np_, TW), jnp.float32),
            jax.ShapeDtypeStruct((np_, HID), jnp.float32),
            jax.ShapeDtypeStruct((np_, HID), jnp.float32),
        ],
    )(pos_pad, wg1, wd1, w2p, b1a, b2a)


def _gather_body(src_hbm, dst_hbm, t_hbm, u_hbm,
                 src_v, dst_v, gs_v, gd_v, u_v, sem):
    epw = src_hbm.shape[0] // NW
    base0 = _wid() * epw

    def chunk(i, carry):
        base = base0 + i * GC
        pltpu.sync_copy(src_hbm.at[pl.ds(base, GC)], src_v)
        pltpu.sync_copy(dst_hbm.at[pl.ds(base, GC)], dst_v)
        pltpu.async_copy(t_hbm.at[src_v], gs_v, sem).wait()
        pltpu.async_copy(t_hbm.at[dst_v], gd_v, sem).wait()

        def row(r, c2):
            for h in range(2):
                u_v[r, pl.ds(h * 16, 16)] = jnp.maximum(
                    gs_v[r, pl.ds(h * 16, 16)]
                    - gd_v[r, pl.ds(HID + h * 16, 16)], 0.0)
            return c2

        lax.fori_loop(0, GC, row, 0, unroll=8)
        pltpu.sync_copy(u_v, u_hbm.at[pl.ds(base, GC)])
        return carry

    lax.fori_loop(0, epw // GC, chunk, 0)


def _sc_gather(src, dst, table, ep):
    mesh = plsc.VectorSubcoreMesh(core_axis_name="c", subcore_axis_name="s")
    f = pl.kernel(
        _gather_body,
        out_type=jax.ShapeDtypeStruct((ep, HID), jnp.float32),
        mesh=mesh,
        scratch_types=[
            pltpu.VMEM((GC,), jnp.int32),
            pltpu.VMEM((GC,), jnp.int32),
            pltpu.VMEM((GC, TW), jnp.float32),
            pltpu.VMEM((GC, TW), jnp.float32),
            pltpu.VMEM((GC, HID), jnp.float32),
            pltpu.SemaphoreType.DMA,
        ],
    )
    return f(src, dst, table)


def _matmul_t_body(u, wb, bbt, mt):
    mt[...] = jax.lax.dot_general(
        wb[...], u[...], (((0,), (1,)), ((), ())),
        preferred_element_type=jnp.float32) + bbt[...]


def _tc_matmul_t(u, wb, bbt):
    e = u.shape[0]
    grid = e // EBLK
    return pl.pallas_call(
        _matmul_t_body,
        grid=(grid,),
        in_specs=[
            pl.BlockSpec((EBLK, HID), lambda i: (i, 0)),
            pl.BlockSpec((HID, HID), lambda i: (0, 0)),
            pl.BlockSpec((HID, 1), lambda i: (0, 0)),
        ],
        out_specs=pl.BlockSpec((HID, EBLK), lambda i: (0, i)),
        out_shape=jax.ShapeDtypeStruct((HID, e), jnp.float32),
    )(u, wb, bbt)


def _scatter_body(dst_hbm, mt_hbm, out_hbm, dst_v, m_v, table):
    ep = dst_hbm.shape[0]
    np_ = table.shape[0]
    feat = _wid()

    zeros = jnp.zeros((16,), jnp.float32)

    def zbody(i, carry):
        table[pl.ds(i * 16, 16)] = zeros
        return carry

    lax.fori_loop(0, np_ // 16, zbody, 0, unroll=8)

    def chunk(ci, carry):
        pltpu.sync_copy(dst_hbm.at[pl.ds(ci * SCC, SCC)], dst_v)
        pltpu.sync_copy(mt_hbm.at[feat, pl.ds(ci * (SCC // 128), SCC // 128)],
                        m_v)

        def grp(r, c2):
            for h in range(8):
                d = dst_v[pl.ds(r * 128 + h * 16, 16)]
                v = m_v[r, pl.ds(h * 16, 16)]
                cur = plsc.load_gather(table, [d])
                plsc.store_scatter(table, [d], jnp.maximum(cur, v))
                chk = plsc.load_gather(table, [d])
                unsat = chk < v

                def cond(mask):
                    return jnp.any(mask)

                def body(mask):
                    cur2 = plsc.load_gather(table, [d], mask=mask)
                    plsc.store_scatter(table, [d],
                                       jnp.maximum(cur2, v), mask=mask)
                    chk2 = plsc.load_gather(table, [d], mask=mask)
                    return jnp.logical_and(mask, chk2 < v)

                lax.while_loop(cond, body, unsat)
            return c2

        lax.fori_loop(0, SCC // 128, grp, 0)
        return carry

    lax.fori_loop(0, ep // SCC, chunk, 0)
    pltpu.sync_copy(table, out_hbm.at[pl.ds(feat * np_, np_)])


def _sc_scatter_max(dst_pad, mt3, np_):
    mesh = plsc.VectorSubcoreMesh(core_axis_name="c", subcore_axis_name="s")
    f = pl.kernel(
        _scatter_body,
        out_type=jax.ShapeDtypeStruct((HID * np_,), jnp.float32),
        mesh=mesh,
        compiler_params=pltpu.CompilerParams(needs_layout_passes=False),
        scratch_types=[
            pltpu.VMEM((SCC,), jnp.int32),
            pltpu.VMEM((SCC // 128, 128), jnp.float32),
            pltpu.VMEM((np_,), jnp.float32),
        ],
    )
    return f(dst_pad, mt3)


def _g2_body(h1t, w2h, p2, d2, t2):
    g2v = jax.lax.dot_general(
        h1t[...], w2h[...], (((0,), (0,)), ((), ())),
        preferred_element_type=jnp.float32) + p2[...]
    pad = jnp.zeros((g2v.shape[0], TW - 2 * HID), jnp.float32)
    t2[...] = jnp.concatenate([g2v, d2[...], pad], axis=1)


def _tc_g2(h1t, w2h, p2, d2):
    np_ = h1t.shape[1]
    grid = np_ // NBLK
    return pl.pallas_call(
        _g2_body,
        grid=(grid,),
        in_specs=[
            pl.BlockSpec((HID, NBLK), lambda i: (0, i)),
            pl.BlockSpec((HID, HID), lambda i: (0, 0)),
            pl.BlockSpec((NBLK, HID), lambda i: (i, 0)),
            pl.BlockSpec((NBLK, HID), lambda i: (i, 0)),
        ],
        out_specs=pl.BlockSpec((NBLK, TW), lambda i: (i, 0)),
        out_shape=jax.ShapeDtypeStruct((np_, TW), jnp.float32),
    )(h1t, w2h, p2, d2)


def _transpose_body(ht, eye, out):
    out[...] = jax.lax.dot_general(
        ht[...], eye[...], (((0,), (0,)), ((), ())),
        preferred_element_type=jnp.float32)


def _tc_transpose(ht):
    np_ = ht.shape[1]
    grid = np_ // NBLK
    eye = jnp.eye(HID, dtype=jnp.float32)
    return pl.pallas_call(
        _transpose_body,
        grid=(grid,),
        in_specs=[
            pl.BlockSpec((HID, NBLK), lambda i: (0, i)),
            pl.BlockSpec((HID, HID), lambda i: (0, 0)),
        ],
        out_specs=pl.BlockSpec((NBLK, HID), lambda i: (i, 0)),
        out_shape=jax.ShapeDtypeStruct((np_, HID), jnp.float32),
    )(ht, eye)


@jax.jit
def kernel(pos, edge_index, batch, W1a, b1a, W1b, b1b, W2a, b2a, W2b, b2b):
    del batch
    src = edge_index[0]
    dst = edge_index[1]
    n = pos.shape[0]
    e = src.shape[0]
    np_ = ((n + NBLK - 1) // NBLK) * NBLK
    lcm = (SCC * EBLK) // 1024
    ep = ((e + lcm - 1) // lcm) * lcm

    pos_pad = jnp.zeros((np_, 3), jnp.float32).at[:n].set(pos)
    dst_pad = jnp.concatenate([dst, jnp.full((ep - e,), n, jnp.int32)])

    wg1 = W1a[0:3] + W1a[3:6]
    wd1 = W1a[3:6]
    w2h = W2a[0:HID]
    w2p = W2a[HID:HID + 3]

    t1, p2, d2 = _tc_tables(pos_pad, wg1, wd1, w2p,
                            b1a.reshape(1, HID), b2a.reshape(1, HID))

    u1 = _sc_gather(src, dst, t1, ep)
    mt1 = _tc_matmul_t(u1, W1b, b1b.reshape(HID, 1))
    h1t = _sc_scatter_max(dst_pad, mt1.reshape(HID, ep // 128, 128), np_)
    h1t = h1t.reshape(HID, np_)

    t2 = _tc_g2(h1t, w2h, p2, d2)
    u2 = _sc_gather(src, dst, t2, ep)
    mt2 = _tc_matmul_t(u2, W2b, b2b.reshape(HID, 1))
    h2t = _sc_scatter_max(dst_pad, mt2.reshape(HID, ep // 128, 128), np_)
    h2t = h2t.reshape(HID, np_)


    return _tc_transpose(h2t)[:n]


# speedup vs baseline: 2.3054x; 1.6019x over previous
import math

import jax
import jax.numpy as jnp
from jax import lax
from jax.experimental import pallas as pl
from jax.experimental.pallas import tpu as pltpu
from jax.experimental.pallas import tpu_sc as plsc

HID = 32
NW = 32
GC = 128
SCC = 4096
NBLK = 1024
EBLK = 7168
TW = 128


def _wid():
    return lax.axis_index("s") * 2 + lax.axis_index("c")


def _tables_body(pos, wg1, wd1, w2p, b1a, b2a, t1, p2, d2):
    p = pos[...]
    g1v = jax.lax.dot_general(p, wg1[...], (((1,), (0,)), ((), ())),
                              preferred_element_type=jnp.float32) + b1a[...]
    d1v = jax.lax.dot_general(p, wd1[...], (((1,), (0,)), ((), ())),
                              preferred_element_type=jnp.float32)
    d2v = jax.lax.dot_general(p, w2p[...], (((1,), (0,)), ((), ())),
                              preferred_element_type=jnp.float32)
    pad = jnp.zeros((p.shape[0], TW - 2 * HID), jnp.float32)
    t1[...] = jnp.concatenate([g1v, d1v, pad], axis=1)
    p2[...] = d2v + b2a[...]
    d2[...] = d2v


def _tc_tables(pos_pad, wg1, wd1, w2p, b1a, b2a):
    np_ = pos_pad.shape[0]
    grid = np_ // NBLK
    return pl.pallas_call(
        _tables_body,
        grid=(grid,),
        in_specs=[
            pl.BlockSpec((NBLK, 3), lambda i: (i, 0)),
            pl.BlockSpec((3, HID), lambda i: (0, 0)),
            pl.BlockSpec((3, HID), lambda i: (0, 0)),
            pl.BlockSpec((3, HID), lambda i: (0, 0)),
            pl.BlockSpec((1, HID), lambda i: (0, 0)),
            pl.BlockSpec((1, HID), lambda i: (0, 0)),
        ],
        out_specs=[
            pl.BlockSpec((NBLK, TW), lambda i: (i, 0)),
            pl.BlockSpec((NBLK, HID), lambda i: (i, 0)),
            pl.BlockSpec((NBLK, HID), lambda i: (i, 0)),
        ],
        out_shape=[
            jax.ShapeDtypeStruct((np_, TW), jnp.float32),
            jax.ShapeDtypeStruct((np_, HID), jnp.float32),
            jax.ShapeDtypeStruct((np_, HID), jnp.float32),
        ],
    )(pos_pad, wg1, wd1, w2p, b1a, b2a)


def _gather_body(src_hbm, dst_hbm, t_hbm, u_hbm,
                 srcp_v, dstp_v, gs0, gd0, gs1, gd1, u0, u1,
                 g0sem, g1sem, usem):
    epw = src_hbm.shape[0] // NW
    base0 = _wid() * epw
    gbufs = ((gs0, gd0, g0sem, u0), (gs1, gd1, g1sem, u1))

    def compute(gs, gd, u):
        def row(r, c2):
            for h in range(2):
                u[r, pl.ds(h * 16, 16)] = jnp.maximum(
                    gs[r, pl.ds(h * 16, 16)]
                    - gd[r, pl.ds(HID + h * 16, 16)], 0.0)
            return c2

        lax.fori_loop(0, GC, row, 0, unroll=8)

    def pair(k, carry):
        base = base0 + k * (2 * GC)
        pltpu.sync_copy(src_hbm.at[pl.ds(base, 2 * GC)], srcp_v)
        pltpu.sync_copy(dst_hbm.at[pl.ds(base, 2 * GC)], dstp_v)
        handles = []
        for b in range(2):
            gs, gd, sem, _ = gbufs[b]
            handles.append((
                pltpu.async_copy(
                    t_hbm.at[srcp_v.at[pl.ds(b * GC, GC)]], gs, sem),
                pltpu.async_copy(
                    t_hbm.at[dstp_v.at[pl.ds(b * GC, GC)]], gd, sem),
            ))
        uh = []
        for b in range(2):
            gs, gd, sem, u = gbufs[b]
            ha, hb = handles[b]
            ha.wait()
            hb.wait()
            compute(gs, gd, u)
            uh.append(pltpu.async_copy(
                u, u_hbm.at[pl.ds(base + b * GC, GC)], usem))
        for h in uh:
            h.wait()
        return carry

    lax.fori_loop(0, epw // (2 * GC), pair, 0)


def _sc_gather(src, dst, table, ep):
    mesh = plsc.VectorSubcoreMesh(core_axis_name="c", subcore_axis_name="s")
    f = pl.kernel(
        _gather_body,
        out_type=jax.ShapeDtypeStruct((ep, HID), jnp.float32),
        mesh=mesh,
        scratch_types=[
            pltpu.VMEM((2 * GC,), jnp.int32),
            pltpu.VMEM((2 * GC,), jnp.int32),
            pltpu.VMEM((GC, TW), jnp.float32),
            pltpu.VMEM((GC, TW), jnp.float32),
            pltpu.VMEM((GC, TW), jnp.float32),
            pltpu.VMEM((GC, TW), jnp.float32),
            pltpu.VMEM((GC, HID), jnp.float32),
            pltpu.VMEM((GC, HID), jnp.float32),
            pltpu.SemaphoreType.DMA,
            pltpu.SemaphoreType.DMA,
            pltpu.SemaphoreType.DMA,
        ],
    )
    return f(src, dst, table)


def _matmul_t_body(u, wb, bbt, mt):
    mt[...] = jax.lax.dot_general(
        wb[...], u[...], (((0,), (1,)), ((), ())),
        preferred_element_type=jnp.float32) + bbt[...]


def _tc_matmul_t(u, wb, bbt):
    e = u.shape[0]
    grid = e // EBLK
    return pl.pallas_call(
        _matmul_t_body,
        grid=(grid,),
        in_specs=[
            pl.BlockSpec((EBLK, HID), lambda i: (i, 0)),
            pl.BlockSpec((HID, HID), lambda i: (0, 0)),
            pl.BlockSpec((HID, 1), lambda i: (0, 0)),
        ],
        out_specs=pl.BlockSpec((HID, EBLK), lambda i: (0, i)),
        out_shape=jax.ShapeDtypeStruct((HID, e), jnp.float32),
    )(u, wb, bbt)


def _scatter_body(dst_hbm, mt_hbm, out_hbm, dst0, m0, dst1, m1,
                  s0sem, s1sem, table):
    ep = dst_hbm.shape[0]
    np_ = table.shape[0]
    feat = _wid()
    nrows = SCC // 128
    bufs = ((dst0, m0, s0sem), (dst1, m1, s1sem))

    zeros = jnp.zeros((16,), jnp.float32)

    def zbody(i, carry):
        table[pl.ds(i * 16, 16)] = zeros
        return carry

    lax.fori_loop(0, np_ // 16, zbody, 0, unroll=8)

    def compute(dst_v, m_v):
        def grp(r, c2):
            unsat_any = jnp.zeros((16,), jnp.bool_)
            for h in range(8):
                d = dst_v[pl.ds(r * 128 + h * 16, 16)]
                v = m_v[r, pl.ds(h * 16, 16)]
                cur = plsc.load_gather(table, [d])
                plsc.store_scatter(table, [d], jnp.maximum(cur, v))
                chk = plsc.load_gather(table, [d])
                unsat_any = jnp.logical_or(unsat_any, chk < v)

            @pl.when(jnp.any(unsat_any))
            def _retry():
                for h in range(8):
                    d = dst_v[pl.ds(r * 128 + h * 16, 16)]
                    v = m_v[r, pl.ds(h * 16, 16)]
                    chk = plsc.load_gather(table, [d])

                    def cond(mask):
                        return jnp.any(mask)

                    def body(mask):
                        cur2 = plsc.load_gather(table, [d], mask=mask)
                        plsc.store_scatter(table, [d],
                                           jnp.maximum(cur2, v), mask=mask)
                        chk2 = plsc.load_gather(table, [d], mask=mask)
                        return jnp.logical_and(mask, chk2 < v)

                    lax.while_loop(cond, body, chk < v)

            return c2

        lax.fori_loop(0, nrows, grp, 0)

    def pair(k, carry):
        handles = []
        for b in range(2):
            dst_v, m_v, sem = bufs[b]
            ci = k * 2 + b
            handles.append((
                pltpu.async_copy(dst_hbm.at[pl.ds(ci * SCC, SCC)], dst_v,
                                 sem),
                pltpu.async_copy(
                    mt_hbm.at[feat, pl.ds(ci * nrows, nrows)], m_v, sem),
            ))
        for b in range(2):
            dst_v, m_v, sem = bufs[b]
            ha, hb = handles[b]
            ha.wait()
            hb.wait()
            compute(dst_v, m_v)
        return carry

    lax.fori_loop(0, ep // (2 * SCC), pair, 0)
    pltpu.sync_copy(table, out_hbm.at[pl.ds(feat * np_, np_)])


def _sc_scatter_max(dst_pad, mt3, np_):
    mesh = plsc.VectorSubcoreMesh(core_axis_name="c", subcore_axis_name="s")
    f = pl.kernel(
        _scatter_body,
        out_type=jax.ShapeDtypeStruct((HID * np_,), jnp.float32),
        mesh=mesh,
        compiler_params=pltpu.CompilerParams(needs_layout_passes=False),
        scratch_types=[
            pltpu.VMEM((SCC,), jnp.int32),
            pltpu.VMEM((SCC // 128, 128), jnp.float32),
            pltpu.VMEM((SCC,), jnp.int32),
            pltpu.VMEM((SCC // 128, 128), jnp.float32),
            pltpu.SemaphoreType.DMA,
            pltpu.SemaphoreType.DMA,
            pltpu.VMEM((np_,), jnp.float32),
        ],
    )
    return f(dst_pad, mt3)


def _g2_body(h1t, w2h, p2, d2, t2):
    g2v = jax.lax.dot_general(
        h1t[...], w2h[...], (((0,), (0,)), ((), ())),
        preferred_element_type=jnp.float32) + p2[...]
    pad = jnp.zeros((g2v.shape[0], TW - 2 * HID), jnp.float32)
    t2[...] = jnp.concatenate([g2v, d2[...], pad], axis=1)


def _tc_g2(h1t, w2h, p2, d2):
    np_ = h1t.shape[1]
    grid = np_ // NBLK
    return pl.pallas_call(
        _g2_body,
        grid=(grid,),
        in_specs=[
            pl.BlockSpec((HID, NBLK), lambda i: (0, i)),
            pl.BlockSpec((HID, HID), lambda i: (0, 0)),
            pl.BlockSpec((NBLK, HID), lambda i: (i, 0)),
            pl.BlockSpec((NBLK, HID), lambda i: (i, 0)),
        ],
        out_specs=pl.BlockSpec((NBLK, TW), lambda i: (i, 0)),
        out_shape=jax.ShapeDtypeStruct((np_, TW), jnp.float32),
    )(h1t, w2h, p2, d2)


def _transpose_body(ht, eye, out):
    out[...] = jax.lax.dot_general(
        ht[...], eye[...], (((0,), (0,)), ((), ())),
        preferred_element_type=jnp.float32)


def _tc_transpose(ht):
    np_ = ht.shape[1]
    grid = np_ // NBLK
    eye = jnp.eye(HID, dtype=jnp.float32)
    return pl.pallas_call(
        _transpose_body,
        grid=(grid,),
        in_specs=[
            pl.BlockSpec((HID, NBLK), lambda i: (0, i)),
            pl.BlockSpec((HID, HID), lambda i: (0, 0)),
        ],
        out_specs=pl.BlockSpec((NBLK, HID), lambda i: (i, 0)),
        out_shape=jax.ShapeDtypeStruct((np_, HID), jnp.float32),
    )(ht, eye)


@jax.jit
def kernel(pos, edge_index, batch, W1a, b1a, W1b, b1b, W2a, b2a, W2b, b2b):
    del batch
    src = edge_index[0]
    dst = edge_index[1]
    n = pos.shape[0]
    e = src.shape[0]
    np_ = ((n + NBLK - 1) // NBLK) * NBLK
    lcm = math.lcm(SCC, EBLK, 2 * GC * NW)
    ep = ((e + lcm - 1) // lcm) * lcm

    pos_pad = jnp.zeros((np_, 3), jnp.float32).at[:n].set(pos)
    dst_pad = jnp.concatenate([dst, jnp.full((ep - e,), n, jnp.int32)])
    src_pad = jnp.concatenate([src, jnp.zeros((ep - e,), jnp.int32)])

    wg1 = W1a[0:3] + W1a[3:6]
    wd1 = W1a[3:6]
    w2h = W2a[0:HID]
    w2p = W2a[HID:HID + 3]

    t1, p2, d2 = _tc_tables(pos_pad, wg1, wd1, w2p,
                            b1a.reshape(1, HID), b2a.reshape(1, HID))

    u1 = _sc_gather(src_pad, dst_pad, t1, ep)
    mt1 = _tc_matmul_t(u1, W1b, b1b.reshape(HID, 1))
    h1t = _sc_scatter_max(dst_pad, mt1.reshape(HID, ep // 128, 128), np_)
    h1t = h1t.reshape(HID, np_)

    t2 = _tc_g2(h1t, w2h, p2, d2)
    u2 = _sc_gather(src_pad, dst_pad, t2, ep)
    mt2 = _tc_matmul_t(u2, W2b, b2b.reshape(HID, 1))
    h2t = _sc_scatter_max(dst_pad, mt2.reshape(HID, ep // 128, 128), np_)
    h2t = h2t.reshape(HID, np_)


    return _tc_transpose(h2t)[:n]
